# baseline (speedup 1.0000x reference)
import functools

import jax
import jax.numpy as jnp
from jax import lax
from jax.experimental import pallas as pl
from jax.experimental.pallas import tpu as pltpu
from jax.experimental.pallas import tpu_sc as plsc

_VOCAB = 100000
_DIM = 64
_BATCH = 4096
_NC = 2
_NS = 16
_NW = _NC * _NS
_ROWS_PER_W = _DIM // _NW


def _embed_gather_t(indices, table_t):
    mesh = plsc.VectorSubcoreMesh(core_axis_name="c", subcore_axis_name="s")

    @functools.partial(
        pl.kernel,
        mesh=mesh,
        compiler_params=pltpu.CompilerParams(
            needs_layout_passes=False,
            skip_device_barrier=True,
            disable_bounds_checks=True,
        ),
        out_type=jax.ShapeDtypeStruct((_DIM, _BATCH), jnp.float32),
        scratch_types=[
            pltpu.VMEM((_BATCH,), jnp.int32),
            pltpu.VMEM((_VOCAB,), jnp.float32),
            pltpu.VMEM((_BATCH,), jnp.float32),
        ],
    )
    def k(idx_hbm, table_hbm, out_hbm, idx_v, row_v, outrow_v):
        wid = lax.axis_index("s") * _NC + lax.axis_index("c")
        pltpu.sync_copy(idx_hbm, idx_v)
        for p in range(_ROWS_PER_W):
            j = p * _NW + wid
            pltpu.sync_copy(table_hbm.at[j], row_v)

            def chunk(c, carry):
                base = c * 64
                for u in range(4):
                    iv = idx_v[pl.ds(base + u * 16, 16)]
                    outrow_v[pl.ds(base + u * 16, 16)] = plsc.load_gather(
                        row_v, [iv]
                    )
                return carry

            lax.fori_loop(0, _BATCH // 64, chunk, 0)
            pltpu.sync_copy(outrow_v, out_hbm.at[j])

    return k(indices, table_t)


def kernel(indices, table):
    out_t = _embed_gather_t(indices.astype(jnp.int32), table.T)
    return out_t.T


# speedup vs baseline: 1.0180x; 1.0180x over previous
import functools

import jax
import jax.numpy as jnp
from jax import lax
from jax.experimental import pallas as pl
from jax.experimental.pallas import tpu as pltpu
from jax.experimental.pallas import tpu_sc as plsc

_VOCAB = 100000
_DIM = 64
_BATCH = 4096
_NC = 2
_NS = 16
_NW = _NC * _NS
_ROWS_PER_W = _DIM // _NW


def _embed_gather_t(indices, table_t):
    mesh = plsc.VectorSubcoreMesh(core_axis_name="c", subcore_axis_name="s")

    @functools.partial(
        pl.kernel,
        mesh=mesh,
        compiler_params=pltpu.CompilerParams(
            needs_layout_passes=False,
            skip_device_barrier=True,
            disable_bounds_checks=True,
        ),
        out_type=jax.ShapeDtypeStruct((_DIM, _BATCH), jnp.float32),
        scratch_types=[
            pltpu.VMEM((_BATCH,), jnp.int32),
            pltpu.VMEM((_VOCAB,), jnp.float32),
            pltpu.VMEM((_BATCH,), jnp.float32),
            pltpu.VMEM((_BATCH,), jnp.float32),
            pltpu.SemaphoreType.DMA,
            pltpu.SemaphoreType.DMA,
        ],
    )
    def k(idx_hbm, table_hbm, out_hbm, idx_v, row_v, outrow_a, outrow_b,
          sem_i, sem_o):
        wid = lax.axis_index("s") * _NC + lax.axis_index("c")
        idx_cp = pltpu.async_copy(idx_hbm, idx_v, sem_i)
        j0 = wid
        j1 = _NW + wid
        pltpu.sync_copy(table_hbm.at[j0], row_v)
        idx_cp.wait()

        def gather_row(outrow_v):
            def chunk(c, carry):
                base = c * 128
                for u in range(8):
                    iv = idx_v[pl.ds(base + u * 16, 16)]
                    outrow_v[pl.ds(base + u * 16, 16)] = plsc.load_gather(
                        row_v, [iv]
                    )
                return carry

            lax.fori_loop(0, _BATCH // 128, chunk, 0)

        gather_row(outrow_a)
        out_cp = pltpu.async_copy(outrow_a, out_hbm.at[j0], sem_o)
        pltpu.sync_copy(table_hbm.at[j1], row_v)
        gather_row(outrow_b)
        out_cp.wait()
        pltpu.sync_copy(outrow_b, out_hbm.at[j1])

    return k(indices, table_t)


def kernel(indices, table):
    out_t = _embed_gather_t(indices.astype(jnp.int32), table.T)
    return out_t.T


# speedup vs baseline: 1.0699x; 1.0510x over previous
import functools

import jax
import jax.numpy as jnp
from jax import lax
from jax.experimental import pallas as pl
from jax.experimental.pallas import tpu as pltpu
from jax.experimental.pallas import tpu_sc as plsc

_VOCAB = 100000
_DIM = 64
_BATCH = 4096
_NC = 2
_NS = 16
_NW = _NC * _NS
_ROWS_PER_W = _DIM // _NW


def _embed_gather_t(indices, table_t):
    mesh = plsc.VectorSubcoreMesh(core_axis_name="c", subcore_axis_name="s")

    @functools.partial(
        pl.kernel,
        mesh=mesh,
        compiler_params=pltpu.CompilerParams(
            needs_layout_passes=False,
            skip_device_barrier=True,
            disable_bounds_checks=True,
        ),
        out_type=jax.ShapeDtypeStruct((_DIM, _BATCH), jnp.float32),
        scratch_types=[
            pltpu.VMEM((_BATCH,), jnp.int32),
            pltpu.VMEM((_VOCAB,), jnp.float32),
            pltpu.VMEM((_BATCH,), jnp.float32),
            pltpu.VMEM((_BATCH,), jnp.float32),
            pltpu.SemaphoreType.DMA,
            pltpu.SemaphoreType.DMA,
        ],
    )
    def k(idx_hbm, table_hbm, out_hbm, idx_v, row_v, outrow_a, outrow_b,
          sem_i, sem_o):
        wid = lax.axis_index("s") * _NC + lax.axis_index("c")
        idx_cp = pltpu.async_copy(idx_hbm, idx_v, sem_i)
        j0 = wid
        j1 = _NW + wid
        pltpu.sync_copy(table_hbm.at[j0], row_v)
        idx_cp.wait()

        def gather_row(outrow_v):
            @plsc.parallel_loop(0, _BATCH, 16, unroll=8)
            def chunk(b):
                iv = idx_v[pl.ds(b, 16)]
                outrow_v[pl.ds(b, 16)] = plsc.load_gather(row_v, [iv])

        gather_row(outrow_a)
        out_cp = pltpu.async_copy(outrow_a, out_hbm.at[j0], sem_o)
        pltpu.sync_copy(table_hbm.at[j1], row_v)
        gather_row(outrow_b)
        out_cp.wait()
        pltpu.sync_copy(outrow_b, out_hbm.at[j1])

    return k(indices, table_t)


def kernel(indices, table):
    out_t = _embed_gather_t(indices.astype(jnp.int32), table.T)
    return out_t.T
